# baseline (speedup 1.0000x reference)
import math

import jax
import jax.numpy as jnp
from jax import lax
from jax.experimental import pallas as pl
from jax.experimental.pallas import tpu as pltpu
from jax.experimental.pallas import tpu_sc as plsc

_N = 8192
_ROWS = 128
_NB = 2048
_NSUB = 4
_CAP = 2048
_RPW = 4
_SPARSITY = 0.05
_K1 = math.ceil(_SPARSITY * _N)
_K2 = _K1 + 1


def _sc_kwta_body(x_hbm, out_hbm, rowa_v, rowb_v, hist_v, cand_v,
                  semi0, semi1, semo0, semo1):
    SIGN = jnp.int32(-2**31)
    IMIN = jnp.int32(-2**31)
    IMAX = jnp.int32(2**31 - 1)
    i16 = jnp.int32(16)
    cid = lax.axis_index("c")
    sid = lax.axis_index("s")
    wid = sid * 2 + cid
    lane = lax.iota(jnp.int32, 16)
    ones = jnp.ones((16,), jnp.int32)
    zero = jnp.zeros((16,), jnp.int32)
    k1v = jnp.full((16,), jnp.int32(_K1), jnp.int32)
    laneoff = (lane & jnp.int32(3)) * jnp.int32(_NB)

    def keys_of(v):
        bi = plsc.bitcast(v, jnp.int32)
        return jnp.where(bi < 0, ~bi ^ SIGN, bi)

    @plsc.parallel_loop(0, _NSUB * _NB // 16, unroll=8)
    def _zeros(i):
        hist_v[pl.ds(i * 16, 16)] = zero

    def compute_row(row_v, row):
        @plsc.parallel_loop(0, _N // 16, unroll=8)
        def _hist(i):
            sk = keys_of(row_v[pl.ds(i * 16, 16)])
            bucket = (sk >> jnp.int32(21)) + jnp.int32(1024) + laneoff
            plsc.addupdate_scatter(hist_v, [bucket], ones)

        @plsc.parallel_loop(
            0, _NB // 16, unroll=4,
            carry=(jnp.int32(0), zero, zero,
                   jnp.full((16,), IMAX, jnp.int32)))
        def _suffix(i, carry):
            tot, nge1_v, na1_v, s1_v = carry
            c = jnp.int32(_NB // 16 - 1) - i
            v = (hist_v[pl.ds(c * 16, 16)] +
                 hist_v[pl.ds(_NB + c * 16, 16)] +
                 hist_v[pl.ds(2 * _NB + c * 16, 16)] +
                 hist_v[pl.ds(3 * _NB + c * 16, 16)])
            hist_v[pl.ds(c * 16, 16)] = zero
            hist_v[pl.ds(_NB + c * 16, 16)] = zero
            hist_v[pl.ds(2 * _NB + c * 16, 16)] = zero
            hist_v[pl.ds(3 * _NB + c * 16, 16)] = zero
            rc = plsc.cumsum(v)
            t = jnp.max(rc)
            s = jnp.full((16,), tot + t, jnp.int32) - rc + v
            ge = s >= k1v
            nge1_v = nge1_v + ge.astype(jnp.int32)
            na1_v = jnp.maximum(na1_v, jnp.where(ge, zero, s))
            s1_v = jnp.minimum(s1_v, jnp.where(ge, s, IMAX))
            return tot + t, nge1_v, na1_v, s1_v
        _, nge1_v, na1_v, s1_v = _suffix
        b1 = jnp.sum(nge1_v) - jnp.int32(1)
        n_above1 = jnp.max(na1_v)
        cnt_b1 = jnp.min(s1_v) - n_above1
        r1 = jnp.int32(_K1) - n_above1
        b1s = b1 - jnp.int32(1024)
        b1sv = jnp.full((16,), b1s, jnp.int32)

        @plsc.parallel_loop(
            0, _N // 16, unroll=8,
            carry=(zero, jnp.full((16,), IMIN, jnp.int32)))
        def _compact(i, carry):
            cntv, mxb_v = carry
            sk = keys_of(row_v[pl.ds(i * 16, 16)])
            bb = sk >> jnp.int32(21)
            m_in = bb == b1sv
            m_bel = bb < b1sv
            mi = m_in.astype(jnp.int32)
            pos = jnp.minimum(cntv + plsc.cumsum(mi) - ones,
                              jnp.int32(_CAP - 1))
            plsc.store_scatter(cand_v, [pos], sk, mask=m_in)
            mxb_v = jnp.maximum(mxb_v, jnp.where(m_bel, sk, IMIN))
            cntv = cntv + plsc.all_reduce_population_count(m_in)
            return cntv, mxb_v
        (_, mxb_v) = _compact
        mx_below = jnp.max(mxb_v)

        ncap = jnp.minimum(cnt_b1, jnp.int32(_CAP))
        nch = (ncap + jnp.int32(15)) // i16
        cntv16 = jnp.full((16,), ncap, jnp.int32)
        kbase = b1s << jnp.int32(21)

        def resolve(nchunks, load):
            def count_ge(ck):
                ckv = jnp.full((16,), ck, jnp.int32)

                def ccb(j, acc):
                    c, dom = load(j)
                    hit = dom & (c >= ckv)
                    return acc + hit.astype(jnp.int32)
                return jnp.sum(lax.fori_loop(0, nchunks, ccb, zero))

            def bbody(i, kk):
                cand_k = kk | (jnp.int32(1) << (jnp.int32(20) - i))
                return jnp.where(count_ge(cand_k) >= r1, cand_k, kk)
            k410 = lax.fori_loop(0, 21, bbody, kbase)
            cnt_at = count_ge(k410)
            kv410 = jnp.full((16,), k410, jnp.int32)

            def mbb(j, mv):
                c, dom = load(j)
                m = dom & (c < kv410)
                return jnp.maximum(mv, jnp.where(m, c, IMIN))
            mxc_v = lax.fori_loop(0, nchunks, mbb,
                                  jnp.full((16,), IMIN, jnp.int32))
            return k410, cnt_at, jnp.max(mxc_v)

        def load_small(j):
            return cand_v[pl.ds(j * 16, 16)], (lane + j * i16) < cntv16

        def load_full(j):
            sk = keys_of(row_v[pl.ds(j * 16, 16)])
            return sk, (sk >> jnp.int32(21)) == b1sv

        def resolve_small():
            return resolve(nch, load_small)

        def resolve_full():
            return resolve(jnp.int32(_N // 16), load_full)

        k410, cnt_at, mx_cand = lax.cond(
            cnt_b1 <= jnp.int32(_CAP), resolve_small, resolve_full)
        k411 = jnp.where(n_above1 + cnt_at >= jnp.int32(_K2), k410,
                         jnp.maximum(mx_below, mx_cand))

        def key_to_f(k):
            fb = jnp.where(k < 0, ~(k ^ SIGN), k)
            return plsc.bitcast(fb, jnp.float32)
        tv = (key_to_f(jnp.full((16,), k410, jnp.int32)) +
              key_to_f(jnp.full((16,), k411, jnp.int32))) * jnp.float32(0.5)

        onef = jnp.full((16,), 1.0, jnp.float32)
        zerof = jnp.zeros((16,), jnp.float32)

        @plsc.parallel_loop(0, _N // 16, unroll=8)
        def _mask(i):
            v = row_v[pl.ds(i * 16, 16)]
            row_v[pl.ds(i * 16, 16)] = jnp.where(v > tv, onef, zerof)

    bufs = (rowa_v, rowb_v)
    sin = (semi0, semi1)
    sout = (semo0, semo1)
    rows = [wid * jnp.int32(_RPW) + jnp.int32(rr) for rr in range(_RPW)]
    pltpu.async_copy(x_hbm.at[rows[0]], bufs[0], sin[0])
    for rr in range(_RPW):
        h = rr & 1
        pltpu.make_async_copy(x_hbm.at[rows[rr]], bufs[h], sin[h]).wait()
        if rr + 1 < _RPW:
            if rr >= 1:
                pltpu.make_async_copy(
                    bufs[1 - h], out_hbm.at[rows[rr - 1]],
                    sout[1 - h]).wait()
            pltpu.async_copy(x_hbm.at[rows[rr + 1]], bufs[1 - h],
                             sin[1 - h])
        compute_row(bufs[h], rows[rr])
        pltpu.async_copy(bufs[h], out_hbm.at[rows[rr]], sout[h])
    pltpu.make_async_copy(bufs[0], out_hbm.at[rows[_RPW - 2]],
                          sout[0]).wait()
    pltpu.make_async_copy(bufs[1], out_hbm.at[rows[_RPW - 1]],
                          sout[1]).wait()


def kernel(x):
    mesh = plsc.VectorSubcoreMesh(core_axis_name="c", subcore_axis_name="s")
    return pl.kernel(
        _sc_kwta_body,
        mesh=mesh,
        out_type=jax.ShapeDtypeStruct((_ROWS, _N), jnp.float32),
        scratch_types=[
            pltpu.VMEM((_N,), jnp.float32),
            pltpu.VMEM((_N,), jnp.float32),
            pltpu.VMEM((_NSUB * _NB,), jnp.int32),
            pltpu.VMEM((_CAP,), jnp.int32),
            pltpu.SemaphoreType.DMA,
            pltpu.SemaphoreType.DMA,
            pltpu.SemaphoreType.DMA,
            pltpu.SemaphoreType.DMA,
        ],
        compiler_params=pltpu.CompilerParams(needs_layout_passes=False),
    )(x)
